# baseline (speedup 1.0000x reference)
import functools

import jax
import jax.numpy as jnp
from jax import lax
from jax.experimental import pallas as pl

K = 256
D = 256
N_FLAT = 25088
BLOCK_R = 3584
GRID = N_FLAT // BLOCK_R


def _vq_body(x_ref, cb_ref, q_ref, loss_ref, cnt_ref):
    i = pl.program_id(0)
    xb = x_ref[:, :]
    cb = cb_ref[:, :]
    dot = lax.dot_general(xb, cb, (((1,), (1,)), ((), ())),
                          preferred_element_type=jnp.float32)
    rowsq = jnp.sum(xb * xb, axis=1, keepdims=True)
    csq = jnp.sum(cb * cb, axis=1)
    dist = (rowsq + csq[None, :]) - 2.0 * dot
    dmin = jnp.min(dist, axis=1, keepdims=True)
    col = lax.broadcasted_iota(jnp.int32, dist.shape, 1)
    idx = jnp.min(jnp.where(dist == dmin, col, jnp.int32(K)), axis=1,
                  keepdims=True)
    onehot = (col == idx).astype(jnp.float32)
    q = lax.dot_general(onehot, cb, (((1,), (0,)), ((), ())),
                        preferred_element_type=jnp.float32)
    q_ref[:, :] = q
    diff = q - xb

    @pl.when(i == 0)
    def _init():
        loss_ref[:, :] = jnp.zeros((1, 1), jnp.float32)
        cnt_ref[:, :] = jnp.zeros((1, K), jnp.float32)

    loss_ref[:, :] += jnp.sum(diff * diff).reshape(1, 1)
    cnt_ref[:, :] += jnp.sum(onehot, axis=0).reshape(1, K)


@functools.partial(jax.jit, static_argnames=())
def _run_vq(flat, codebook):
    return pl.pallas_call(
        _vq_body,
        grid=(GRID,),
        in_specs=[
            pl.BlockSpec((BLOCK_R, D), lambda i: (i, 0)),
            pl.BlockSpec((K, D), lambda i: (0, 0)),
        ],
        out_specs=[
            pl.BlockSpec((BLOCK_R, D), lambda i: (i, 0)),
            pl.BlockSpec((1, 1), lambda i: (0, 0)),
            pl.BlockSpec((1, K), lambda i: (0, 0)),
        ],
        out_shape=[
            jax.ShapeDtypeStruct((N_FLAT, D), jnp.float32),
            jax.ShapeDtypeStruct((1, 1), jnp.float32),
            jax.ShapeDtypeStruct((1, K), jnp.float32),
        ],
    )(flat, codebook)


def _conv(x, w, b, stride, padding):
    y = lax.conv_general_dilated(x, w, (stride, stride),
                                 ((padding, padding), (padding, padding)),
                                 dimension_numbers=('NCHW', 'OIHW', 'NCHW'))
    return y + b[None, :, None, None]


def _conv_t(x, w, b, stride, padding):
    k = w.shape[2]
    w_t = jnp.transpose(jnp.flip(w, (2, 3)), (1, 0, 2, 3))
    pad = k - 1 - padding
    y = lax.conv_general_dilated(x.astype(jnp.bfloat16),
                                 w_t.astype(jnp.bfloat16), (1, 1),
                                 ((pad, pad), (pad, pad)),
                                 lhs_dilation=(stride, stride),
                                 dimension_numbers=('NCHW', 'OIHW', 'NCHW'),
                                 preferred_element_type=jnp.float32)
    return y + b[None, :, None, None]


def kernel(x, enc_w1, enc_b1, enc_w2, enc_b2, enc_w3, enc_b3, codebook,
           dec_w1, dec_b1, dec_w2, dec_b2, dec_w3, dec_b3):
    h = jax.nn.relu(_conv(x, enc_w1, enc_b1, 2, 1))
    h = jax.nn.relu(_conv(h, enc_w2, enc_b2, 2, 1))
    encoded = _conv(h, enc_w3, enc_b3, 1, 0)

    flat = encoded.reshape(-1, D)
    q_flat, loss_sum, counts = _run_vq(flat, codebook)
    quantized = q_flat.reshape(encoded.shape)
    m = loss_sum[0, 0] / flat.size
    vq_loss = m + 0.25 * m
    avg_probs = counts[0] / flat.shape[0]
    perplexity = jnp.exp(-jnp.sum(avg_probs * jnp.log(avg_probs + 1e-10)))

    d = jax.nn.relu(_conv_t(quantized, dec_w1, dec_b1, 2, 1))
    d = jax.nn.relu(_conv_t(d, dec_w2, dec_b2, 2, 1))
    decoded = jax.nn.sigmoid(_conv_t(d, dec_w3, dec_b3, 1, 0))
    return (decoded, encoded, quantized, vq_loss, perplexity)


# speedup vs baseline: 2.3665x; 2.3665x over previous
import functools

import jax
import jax.numpy as jnp
from jax import lax
from jax.experimental import pallas as pl

K = 256
D = 256
N_FLAT = 25088
BLOCK_R = 3584
GRID = N_FLAT // BLOCK_R


def _vq_body(x_ref, cb_ref, q_ref, loss_ref, cnt_ref):
    i = pl.program_id(0)
    xb = x_ref[:, :]
    cb = cb_ref[:, :]
    dot = lax.dot_general(xb, cb, (((1,), (1,)), ((), ())),
                          preferred_element_type=jnp.float32)
    rowsq = jnp.sum(xb * xb, axis=1, keepdims=True)
    csq = jnp.sum(cb * cb, axis=1)
    dist = (rowsq + csq[None, :]) - 2.0 * dot
    dmin = jnp.min(dist, axis=1, keepdims=True)
    col = lax.broadcasted_iota(jnp.int32, dist.shape, 1)
    idx = jnp.min(jnp.where(dist == dmin, col, jnp.int32(K)), axis=1,
                  keepdims=True)
    onehot = (col == idx).astype(jnp.float32)
    q = lax.dot_general(onehot, cb, (((1,), (0,)), ((), ())),
                        preferred_element_type=jnp.float32)
    q_ref[:, :] = q
    diff = q - xb

    @pl.when(i == 0)
    def _init():
        loss_ref[:, :] = jnp.zeros((1, 1), jnp.float32)
        cnt_ref[:, :] = jnp.zeros((1, K), jnp.float32)

    loss_ref[:, :] += jnp.sum(diff * diff).reshape(1, 1)
    cnt_ref[:, :] += jnp.sum(onehot, axis=0).reshape(1, K)


@functools.partial(jax.jit, static_argnames=())
def _run_vq(flat, codebook):
    return pl.pallas_call(
        _vq_body,
        grid=(GRID,),
        in_specs=[
            pl.BlockSpec((BLOCK_R, D), lambda i: (i, 0)),
            pl.BlockSpec((K, D), lambda i: (0, 0)),
        ],
        out_specs=[
            pl.BlockSpec((BLOCK_R, D), lambda i: (i, 0)),
            pl.BlockSpec((1, 1), lambda i: (0, 0)),
            pl.BlockSpec((1, K), lambda i: (0, 0)),
        ],
        out_shape=[
            jax.ShapeDtypeStruct((N_FLAT, D), jnp.float32),
            jax.ShapeDtypeStruct((1, 1), jnp.float32),
            jax.ShapeDtypeStruct((1, K), jnp.float32),
        ],
    )(flat, codebook)


def _conv_s2k4p1(x, w, b):
    n, c, h, _ = x.shape
    s = h // 2
    xr = x.reshape(n, c, s, 2, s, 2)
    kys = [[1, 3], [0, 2]]
    phases, kparts = [], []
    for py in range(2):
        for px in range(2):
            xp = xr[:, :, :, py, :, px]
            padv = (0, 1) if py == 0 else (1, 0)
            padh = (0, 1) if px == 0 else (1, 0)
            phases.append(jnp.pad(xp, ((0, 0), (0, 0), padv, padh)))
            kparts.append(w[:, :, kys[py], :][:, :, :, kys[px]])
    xcat = jnp.concatenate(phases, axis=1)
    kcat = jnp.concatenate(kparts, axis=1)
    y = lax.conv_general_dilated(xcat, kcat, (1, 1), ((0, 0), (0, 0)),
                                 dimension_numbers=('NCHW', 'OIHW', 'NCHW'))
    return y + b[None, :, None, None]


def _conv_t_s2k4p1(x, w, b):
    n, cin, s, _ = x.shape
    cout = w.shape[1]
    wt = jnp.transpose(w, (1, 0, 2, 3))
    xp = jnp.pad(x, ((0, 0), (0, 0), (1, 1), (1, 1)))
    kys = [[3, 1], [2, 0]]
    kparts = [wt[:, :, kys[p], :][:, :, :, kys[q]]
              for p in range(2) for q in range(2)]
    kcat = jnp.concatenate(kparts, axis=0)
    y = lax.conv_general_dilated(xp, kcat, (1, 1), ((0, 0), (0, 0)),
                                 dimension_numbers=('NCHW', 'OIHW', 'NCHW'))
    parts = [y[:, (p * 2 + q) * cout:(p * 2 + q + 1) * cout,
               p:p + s, q:q + s] for p in range(2) for q in range(2)]
    y6 = jnp.stack(parts).reshape(2, 2, n, cout, s, s)
    out = y6.transpose(2, 3, 4, 0, 5, 1).reshape(n, cout, 2 * s, 2 * s)
    return out + b[None, :, None, None]


def _conv(x, w, b, stride, padding):
    y = lax.conv_general_dilated(x, w, (stride, stride),
                                 ((padding, padding), (padding, padding)),
                                 dimension_numbers=('NCHW', 'OIHW', 'NCHW'))
    return y + b[None, :, None, None]


def _conv_t(x, w, b, stride, padding):
    k = w.shape[2]
    w_t = jnp.transpose(jnp.flip(w, (2, 3)), (1, 0, 2, 3))
    pad = k - 1 - padding
    y = lax.conv_general_dilated(x, w_t, (1, 1),
                                 ((pad, pad), (pad, pad)),
                                 lhs_dilation=(stride, stride),
                                 dimension_numbers=('NCHW', 'OIHW', 'NCHW'))
    return y + b[None, :, None, None]


def kernel(x, enc_w1, enc_b1, enc_w2, enc_b2, enc_w3, enc_b3, codebook,
           dec_w1, dec_b1, dec_w2, dec_b2, dec_w3, dec_b3):
    h = jax.nn.relu(_conv_s2k4p1(x, enc_w1, enc_b1))
    h = jax.nn.relu(_conv_s2k4p1(h, enc_w2, enc_b2))
    encoded = _conv(h, enc_w3, enc_b3, 1, 0)

    flat = encoded.reshape(-1, D)
    q_flat, loss_sum, counts = _run_vq(flat, codebook)
    quantized = q_flat.reshape(encoded.shape)
    m = loss_sum[0, 0] / flat.size
    vq_loss = m + 0.25 * m
    avg_probs = counts[0] / flat.shape[0]
    perplexity = jnp.exp(-jnp.sum(avg_probs * jnp.log(avg_probs + 1e-10)))

    d = jax.nn.relu(_conv_t_s2k4p1(quantized, dec_w1, dec_b1))
    d = jax.nn.relu(_conv_t_s2k4p1(d, dec_w2, dec_b2))
    decoded = jax.nn.sigmoid(_conv_t(d, dec_w3, dec_b3, 1, 0))
    return (decoded, encoded, quantized, vq_loss, perplexity)


# speedup vs baseline: 2.9817x; 1.2600x over previous
import functools

import jax
import jax.numpy as jnp
from jax import lax
from jax.experimental import pallas as pl

K = 256
D = 256
N_FLAT = 25088
BLOCK_R = 3584
GRID = N_FLAT // BLOCK_R


def _vq_body(x_ref, cb_ref, q_ref, loss_ref, cnt_ref):
    i = pl.program_id(0)
    xb = x_ref[:, :]
    cb = cb_ref[:, :]
    dot = lax.dot_general(xb, cb, (((1,), (1,)), ((), ())),
                          preferred_element_type=jnp.float32)
    rowsq = jnp.sum(xb * xb, axis=1, keepdims=True)
    csq = jnp.sum(cb * cb, axis=1)
    dist = (rowsq + csq[None, :]) - 2.0 * dot
    dmin = jnp.min(dist, axis=1, keepdims=True)
    col = lax.broadcasted_iota(jnp.int32, dist.shape, 1)
    idx = jnp.min(jnp.where(dist == dmin, col, jnp.int32(K)), axis=1,
                  keepdims=True)
    onehot = (col == idx).astype(jnp.float32)
    q = lax.dot_general(onehot, cb, (((1,), (0,)), ((), ())),
                        preferred_element_type=jnp.float32)
    q_ref[:, :] = q
    diff = q - xb

    @pl.when(i == 0)
    def _init():
        loss_ref[:, :] = jnp.zeros((1, 1), jnp.float32)
        cnt_ref[:, :] = jnp.zeros((1, K), jnp.float32)

    loss_ref[:, :] += jnp.sum(diff * diff).reshape(1, 1)
    cnt_ref[:, :] += jnp.sum(onehot, axis=0).reshape(1, K)


@functools.partial(jax.jit, static_argnames=())
def _run_vq(flat, codebook):
    return pl.pallas_call(
        _vq_body,
        grid=(GRID,),
        in_specs=[
            pl.BlockSpec((BLOCK_R, D), lambda i: (i, 0)),
            pl.BlockSpec((K, D), lambda i: (0, 0)),
        ],
        out_specs=[
            pl.BlockSpec((BLOCK_R, D), lambda i: (i, 0)),
            pl.BlockSpec((1, 1), lambda i: (0, 0)),
            pl.BlockSpec((1, K), lambda i: (0, 0)),
        ],
        out_shape=[
            jax.ShapeDtypeStruct((N_FLAT, D), jnp.float32),
            jax.ShapeDtypeStruct((1, 1), jnp.float32),
            jax.ShapeDtypeStruct((1, K), jnp.float32),
        ],
    )(flat, codebook)


def _conv_s2k4p1(x, w, b):
    n, c, h, _ = x.shape
    s = h // 2
    xr = x.reshape(n, c, s, 2, s, 2)
    kys = [[1, 3], [0, 2]]
    phases, kparts = [], []
    for py in range(2):
        for px in range(2):
            xp = xr[:, :, :, py, :, px]
            padv = (0, 1) if py == 0 else (1, 0)
            padh = (0, 1) if px == 0 else (1, 0)
            phases.append(jnp.pad(xp, ((0, 0), (0, 0), padv, padh)))
            kparts.append(w[:, :, kys[py], :][:, :, :, kys[px]])
    xcat = jnp.concatenate(phases, axis=1)
    kcat = jnp.concatenate(kparts, axis=1)
    y = lax.conv_general_dilated(xcat, kcat, (1, 1), ((0, 0), (0, 0)),
                                 dimension_numbers=('NCHW', 'OIHW', 'NCHW'))
    return y + b[None, :, None, None]


def _conv_t_s2k4p1(x, w, b):
    n, cin, s, _ = x.shape
    cout = w.shape[1]
    wt = jnp.transpose(w, (1, 0, 2, 3))
    xp = jnp.pad(x, ((0, 0), (0, 0), (1, 1), (1, 1)))
    kys = [[3, 1], [2, 0]]
    kparts = [wt[:, :, kys[p], :][:, :, :, kys[q]]
              for p in range(2) for q in range(2)]
    kcat = jnp.concatenate(kparts, axis=0)
    y = lax.conv_general_dilated(xp, kcat, (1, 1), ((0, 0), (0, 0)),
                                 dimension_numbers=('NCHW', 'OIHW', 'NCHW'))
    parts = [y[:, (p * 2 + q) * cout:(p * 2 + q + 1) * cout,
               p:p + s, q:q + s] for p in range(2) for q in range(2)]
    y6 = jnp.stack(parts).reshape(2, 2, n, cout, s, s)
    out = y6.transpose(2, 3, 4, 0, 5, 1).reshape(n, cout, 2 * s, 2 * s)
    return out + b[None, :, None, None]


def _conv(x, w, b, stride, padding):
    y = lax.conv_general_dilated(x, w, (stride, stride),
                                 ((padding, padding), (padding, padding)),
                                 dimension_numbers=('NCHW', 'OIHW', 'NCHW'))
    return y + b[None, :, None, None]


def _conv_t(x, w, b, stride, padding):
    k = w.shape[2]
    w_t = jnp.transpose(jnp.flip(w, (2, 3)), (1, 0, 2, 3))
    pad = k - 1 - padding
    y = lax.conv_general_dilated(x, w_t, (1, 1),
                                 ((pad, pad), (pad, pad)),
                                 lhs_dilation=(stride, stride),
                                 dimension_numbers=('NCHW', 'OIHW', 'NCHW'))
    return y + b[None, :, None, None]


def kernel(x, enc_w1, enc_b1, enc_w2, enc_b2, enc_w3, enc_b3, codebook,
           dec_w1, dec_b1, dec_w2, dec_b2, dec_w3, dec_b3):
    h = jax.nn.relu(_conv(x, enc_w1, enc_b1, 2, 1))
    h = jax.nn.relu(_conv(h, enc_w2, enc_b2, 2, 1))
    encoded = _conv(h, enc_w3, enc_b3, 1, 0)

    flat = encoded.reshape(-1, D)
    q_flat, loss_sum, counts = _run_vq(flat, codebook)
    quantized = q_flat.reshape(encoded.shape)
    m = loss_sum[0, 0] / flat.size
    vq_loss = m + 0.25 * m
    avg_probs = counts[0] / flat.shape[0]
    perplexity = jnp.exp(-jnp.sum(avg_probs * jnp.log(avg_probs + 1e-10)))

    d = jax.nn.relu(_conv_t_s2k4p1(quantized, dec_w1, dec_b1))
    d = jax.nn.relu(_conv_t_s2k4p1(d, dec_w2, dec_b2))
    decoded = jax.nn.sigmoid(_conv_t(d, dec_w3, dec_b3, 1, 0))
    return (decoded, encoded, quantized, vq_loss, perplexity)


# speedup vs baseline: 3.0873x; 1.0354x over previous
import functools

import jax
import jax.numpy as jnp
from jax import lax
from jax.experimental import pallas as pl
from jax.experimental.pallas import tpu as pltpu
from jax.experimental.pallas import tpu_sc as plsc

K = 256
D = 256
N_FLAT = 25088
BLOCK_R = 3584
GRID = N_FLAT // BLOCK_R

NC = 2
NS = 16
NW = NC * NS
B_PER_W = N_FLAT // NW
CHUNK = 392
N_CHUNKS = B_PER_W // CHUNK


def _argmin_body(x_ref, cb_ref, idx_ref, loss_ref):
    i = pl.program_id(0)
    xb = x_ref[:, :]
    cb = cb_ref[:, :]
    dot = lax.dot_general(xb, cb, (((1,), (1,)), ((), ())),
                          preferred_element_type=jnp.float32)
    rowsq = jnp.sum(xb * xb, axis=1, keepdims=True)
    csq = jnp.sum(cb * cb, axis=1)
    dist = (rowsq + csq[None, :]) - 2.0 * dot
    dmin = jnp.min(dist, axis=1, keepdims=True)
    col = lax.broadcasted_iota(jnp.int32, dist.shape, 1)
    idx = jnp.min(jnp.where(dist == dmin, col, jnp.int32(K)), axis=1,
                  keepdims=True)
    idx_ref[...] = jnp.transpose(idx).reshape(1, 1, BLOCK_R)

    @pl.when(i == 0)
    def _init():
        loss_ref[:, :] = jnp.zeros((1, 1), jnp.float32)

    loss_ref[:, :] += jnp.sum(dmin).reshape(1, 1)


@functools.partial(jax.jit, static_argnames=())
def _run_argmin(flat, codebook):
    return pl.pallas_call(
        _argmin_body,
        grid=(GRID,),
        in_specs=[
            pl.BlockSpec((BLOCK_R, D), lambda i: (i, 0)),
            pl.BlockSpec((K, D), lambda i: (0, 0)),
        ],
        out_specs=[
            pl.BlockSpec((1, 1, BLOCK_R), lambda i: (i, 0, 0)),
            pl.BlockSpec((1, 1), lambda i: (0, 0)),
        ],
        out_shape=[
            jax.ShapeDtypeStruct((GRID, 1, BLOCK_R), jnp.int32),
            jax.ShapeDtypeStruct((1, 1), jnp.float32),
        ],
    )(flat, codebook)


def _gather_body(cb_hbm, idx_hbm, ones_hbm, zeros_hbm,
                 out_hbm, cnt_hbm, idx_v, rows_v, ones_v, cnt_sh, sem):
    cid = lax.axis_index("c")
    sid = lax.axis_index("s")
    wid = sid * NC + cid
    base = wid * B_PER_W
    pltpu.sync_copy(idx_hbm.at[pl.ds(base, B_PER_W)], idx_v)
    for ch in range(N_CHUNKS):
        pltpu.async_copy(cb_hbm.at[idx_v.at[pl.ds(ch * CHUNK, CHUNK)]],
                         rows_v, sem).wait()
        pltpu.sync_copy(rows_v, out_hbm.at[pl.ds(base + ch * CHUNK, CHUNK)])
    pltpu.sync_copy(ones_hbm.at[pl.ds(0, B_PER_W)], ones_v)

    @pl.when(sid == 0)
    def _init():
        pltpu.sync_copy(zeros_hbm, cnt_sh)

    plsc.subcore_barrier()
    pltpu.sync_copy(ones_v, cnt_sh.at[idx_v], add=True)
    plsc.subcore_barrier()

    @pl.when(sid == 0)
    def _pub():
        pltpu.sync_copy(cnt_sh, cnt_hbm.at[cid])


@functools.partial(jax.jit, static_argnames=())
def _run_gather(codebook, idx_flat, ones, zeros):
    mesh = plsc.VectorSubcoreMesh(core_axis_name="c", subcore_axis_name="s")
    kfn = functools.partial(
        pl.kernel, mesh=mesh,
        out_type=[
            jax.ShapeDtypeStruct((N_FLAT, D), jnp.float32),
            jax.ShapeDtypeStruct((NC, K), jnp.float32),
        ],
        scratch_types=[
            pltpu.VMEM((B_PER_W,), jnp.int32),
            pltpu.VMEM((CHUNK, D), jnp.float32),
            pltpu.VMEM((B_PER_W,), jnp.float32),
            pltpu.VMEM_SHARED((K,), jnp.float32),
            pltpu.SemaphoreType.DMA,
        ],
    )(_gather_body)
    return kfn(codebook, idx_flat, ones, zeros)


def _conv(x, w, b, stride, padding):
    y = lax.conv_general_dilated(x, w, (stride, stride),
                                 ((padding, padding), (padding, padding)),
                                 dimension_numbers=('NCHW', 'OIHW', 'NCHW'))
    return y + b[None, :, None, None]


def _conv_t(x, w, b, stride, padding):
    k = w.shape[2]
    w_t = jnp.transpose(jnp.flip(w, (2, 3)), (1, 0, 2, 3))
    pad = k - 1 - padding
    y = lax.conv_general_dilated(x, w_t, (1, 1),
                                 ((pad, pad), (pad, pad)),
                                 lhs_dilation=(stride, stride),
                                 dimension_numbers=('NCHW', 'OIHW', 'NCHW'))
    return y + b[None, :, None, None]


def kernel(x, enc_w1, enc_b1, enc_w2, enc_b2, enc_w3, enc_b3, codebook,
           dec_w1, dec_b1, dec_w2, dec_b2, dec_w3, dec_b3):
    h = jax.nn.relu(_conv(x, enc_w1, enc_b1, 2, 1))
    h = jax.nn.relu(_conv(h, enc_w2, enc_b2, 2, 1))
    encoded = _conv(h, enc_w3, enc_b3, 1, 0)

    flat = encoded.reshape(-1, D)
    idx3, loss_sum = _run_argmin(flat, codebook)
    idx_flat = idx3.reshape(N_FLAT)
    ones = jnp.ones((N_FLAT,), jnp.float32)
    zeros = jnp.zeros((K,), jnp.float32)
    q_flat, counts2 = _run_gather(codebook, idx_flat, ones, zeros)
    quantized = q_flat.reshape(encoded.shape)
    m = loss_sum[0, 0] / encoded.size
    vq_loss = m + 0.25 * m
    avg_probs = (counts2[0] + counts2[1]) / N_FLAT
    perplexity = jnp.exp(-jnp.sum(avg_probs * jnp.log(avg_probs + 1e-10)))

    d = jax.nn.relu(_conv_t(quantized, dec_w1, dec_b1, 2, 1))
    d = jax.nn.relu(_conv_t(d, dec_w2, dec_b2, 2, 1))
    decoded = jax.nn.sigmoid(_conv_t(d, dec_w3, dec_b3, 1, 0))
    return (decoded, encoded, quantized, vq_loss, perplexity)


# speedup vs baseline: 3.8339x; 1.2419x over previous
import functools

import jax
import jax.numpy as jnp
from jax import lax
from jax.experimental import pallas as pl
from jax.experimental.pallas import tpu as pltpu
from jax.experimental.pallas import tpu_sc as plsc

K = 256
D = 256
N_FLAT = 25088
BLOCK_R = 3584
GRID = N_FLAT // BLOCK_R

NC = 2
NS = 16
NW = NC * NS
B_PER_W = N_FLAT // NW


def _vq_body(x_ref, cb_ref, q_ref, idx_ref, loss_ref):
    i = pl.program_id(0)
    xb = x_ref[:, :]
    cb = cb_ref[:, :]
    dot = lax.dot_general(xb, cb, (((1,), (1,)), ((), ())),
                          preferred_element_type=jnp.float32)
    rowsq = jnp.sum(xb * xb, axis=1, keepdims=True)
    csq = jnp.sum(cb * cb, axis=1)
    dist = (rowsq + csq[None, :]) - 2.0 * dot
    dmin = jnp.min(dist, axis=1, keepdims=True)
    col = lax.broadcasted_iota(jnp.int32, dist.shape, 1)
    idx = jnp.min(jnp.where(dist == dmin, col, jnp.int32(K)), axis=1,
                  keepdims=True)
    onehot = (col == idx).astype(jnp.float32)
    q = lax.dot_general(onehot, cb, (((1,), (0,)), ((), ())),
                        preferred_element_type=jnp.float32)
    q_ref[:, :] = q
    idx_ref[...] = jnp.transpose(idx).reshape(1, 1, BLOCK_R)
    diff = q - xb

    @pl.when(i == 0)
    def _init():
        loss_ref[:, :] = jnp.zeros((1, 1), jnp.float32)

    loss_ref[:, :] += jnp.sum(diff * diff).reshape(1, 1)


@functools.partial(jax.jit, static_argnames=())
def _run_vq(flat, codebook):
    return pl.pallas_call(
        _vq_body,
        grid=(GRID,),
        in_specs=[
            pl.BlockSpec((BLOCK_R, D), lambda i: (i, 0)),
            pl.BlockSpec((K, D), lambda i: (0, 0)),
        ],
        out_specs=[
            pl.BlockSpec((BLOCK_R, D), lambda i: (i, 0)),
            pl.BlockSpec((1, 1, BLOCK_R), lambda i: (i, 0, 0)),
            pl.BlockSpec((1, 1), lambda i: (0, 0)),
        ],
        out_shape=[
            jax.ShapeDtypeStruct((N_FLAT, D), jnp.float32),
            jax.ShapeDtypeStruct((GRID, 1, BLOCK_R), jnp.int32),
            jax.ShapeDtypeStruct((1, 1), jnp.float32),
        ],
    )(flat, codebook)


def _hist_body(idx_hbm, ones_hbm, zeros_hbm, cnt_hbm, idx_v, ones_v, cnt_sh):
    cid = lax.axis_index("c")
    sid = lax.axis_index("s")
    wid = sid * NC + cid
    base = wid * B_PER_W
    pltpu.sync_copy(idx_hbm.at[pl.ds(base, B_PER_W)], idx_v)
    pltpu.sync_copy(ones_hbm.at[pl.ds(0, B_PER_W)], ones_v)

    @pl.when(sid == 0)
    def _init():
        pltpu.sync_copy(zeros_hbm, cnt_sh)

    plsc.subcore_barrier()
    pltpu.sync_copy(ones_v, cnt_sh.at[idx_v], add=True)
    plsc.subcore_barrier()

    @pl.when(sid == 0)
    def _pub():
        pltpu.sync_copy(cnt_sh, cnt_hbm.at[cid])


@functools.partial(jax.jit, static_argnames=())
def _run_hist(idx_flat, ones, zeros):
    mesh = plsc.VectorSubcoreMesh(core_axis_name="c", subcore_axis_name="s")
    kfn = functools.partial(
        pl.kernel, mesh=mesh,
        out_type=jax.ShapeDtypeStruct((NC, K), jnp.float32),
        scratch_types=[
            pltpu.VMEM((B_PER_W,), jnp.int32),
            pltpu.VMEM((B_PER_W,), jnp.float32),
            pltpu.VMEM_SHARED((K,), jnp.float32),
        ],
    )(_hist_body)
    return kfn(idx_flat, ones, zeros)


def _conv(x, w, b, stride, padding):
    y = lax.conv_general_dilated(x, w, (stride, stride),
                                 ((padding, padding), (padding, padding)),
                                 dimension_numbers=('NCHW', 'OIHW', 'NCHW'))
    return y + b[None, :, None, None]


def _conv_t(x, w, b, stride, padding):
    k = w.shape[2]
    w_t = jnp.transpose(jnp.flip(w, (2, 3)), (1, 0, 2, 3))
    pad = k - 1 - padding
    y = lax.conv_general_dilated(x, w_t, (1, 1),
                                 ((pad, pad), (pad, pad)),
                                 lhs_dilation=(stride, stride),
                                 dimension_numbers=('NCHW', 'OIHW', 'NCHW'))
    return y + b[None, :, None, None]


def kernel(x, enc_w1, enc_b1, enc_w2, enc_b2, enc_w3, enc_b3, codebook,
           dec_w1, dec_b1, dec_w2, dec_b2, dec_w3, dec_b3):
    h = jax.nn.relu(_conv(x, enc_w1, enc_b1, 2, 1))
    h = jax.nn.relu(_conv(h, enc_w2, enc_b2, 2, 1))
    encoded = _conv(h, enc_w3, enc_b3, 1, 0)

    flat = encoded.reshape(-1, D)
    q_flat, idx3, loss_sum = _run_vq(flat, codebook)
    quantized = q_flat.reshape(encoded.shape)
    idx_flat = idx3.reshape(N_FLAT)
    ones = jnp.ones((N_FLAT,), jnp.float32)
    zeros = jnp.zeros((K,), jnp.float32)
    counts2 = _run_hist(idx_flat, ones, zeros)
    m = loss_sum[0, 0] / encoded.size
    vq_loss = m + 0.25 * m
    avg_probs = (counts2[0] + counts2[1]) / N_FLAT
    perplexity = jnp.exp(-jnp.sum(avg_probs * jnp.log(avg_probs + 1e-10)))

    d = jax.nn.relu(_conv_t(quantized, dec_w1, dec_b1, 2, 1))
    d = jax.nn.relu(_conv_t(d, dec_w2, dec_b2, 2, 1))
    decoded = jax.nn.sigmoid(_conv_t(d, dec_w3, dec_b3, 1, 0))
    return (decoded, encoded, quantized, vq_loss, perplexity)
